# baseline (speedup 1.0000x reference)
import jax
import jax.numpy as jnp
from jax import lax
from jax.experimental import pallas as pl
from jax.experimental.pallas import tpu as pltpu
from jax.experimental.pallas import tpu_sc as plsc

B, T, G, D, V = 64, 10, 900, 128, 11
TG = T * G
NC, NS = 2, 16
NW = NC * NS
BPW = B // NW
CH = (G + 15) // 16
GP = CH * 16
LN2 = 0.6931471805599453
MSE_BIAS = G * D * 1e-6 + 1.0


def _tables_body(tbl_ref, w_ref, s2_ref, m2_ref):
    t = tbl_ref[...]
    wv = w_ref[...]
    s = jnp.sum(t * wv, axis=1)
    s2_ref[...] = s[:, None] + s[None, :]
    gram = lax.dot_general(t, t, (((1,), (1,)), ((), ())),
                           preferred_element_type=jnp.float32)
    nrm = jnp.sum(t * t, axis=1)
    m2_ref[...] = nrm[:, None] + nrm[None, :] - 2.0 * gram


def _sc_body(dag_hbm, term_hbm, maskf_hbm, gum_hbm, s2_hbm, m2_hbm,
             out_hbm,
             dag_v, term_v, mask_v, gum_v, logit_v, s2_v, m2_v,
             out_st, sem):
    wid = lax.axis_index("s") * NC + lax.axis_index("c")
    iota = lax.broadcasted_iota(jnp.int32, (16,), 0)
    zf = jnp.zeros((16,), jnp.float32)
    zi = jnp.zeros((16,), jnp.int32)

    cps = [
        pltpu.async_copy(s2_hbm, s2_v, sem),
        pltpu.async_copy(m2_hbm, m2_v, sem),
        pltpu.async_copy(dag_hbm.at[pl.ds(wid * (BPW * TG), BPW * TG)],
                         dag_v.at[pl.ds(0, BPW * TG)], sem),
        pltpu.async_copy(term_hbm.at[pl.ds(wid * (BPW * G), BPW * G)],
                         term_v.at[pl.ds(0, BPW * G)], sem),
        pltpu.async_copy(maskf_hbm.at[pl.ds(wid * (BPW * G), BPW * G)],
                         mask_v.at[pl.ds(0, BPW * G)], sem),
        pltpu.async_copy(gum_hbm.at[pl.ds(wid * (BPW * G), BPW * G)],
                         gum_v.at[pl.ds(0, BPW * G)], sem),
    ]
    for cp in cps:
        cp.wait()
    dag_v[pl.ds(BPW * TG, 16)] = zi
    t_tail = term_v[pl.ds(BPW * G - 4, 16)]
    term_v[pl.ds(BPW * G - 4, 16)] = jnp.where(iota < 4, t_tail, 0)

    for j in range(BPW):
        b = wid * BPW + j
        doff = j * TG
        poff = j * G

        def chunk_body(c, carry):
            bs, bi, se, ms = carry
            goff = c * 16
            gidx = goff + iota
            valid = gidx < G
            tok0 = dag_v[pl.ds(doff + goff, 16)]
            tok1 = dag_v[pl.ds(doff + G + goff, 16)]
            acc = plsc.load_gather(s2_v, [tok0 * 16 + tok1])
            for t in range(2, T, 2):
                ta = dag_v[pl.ds(doff + t * G + goff, 16)]
                tb = dag_v[pl.ds(doff + (t + 1) * G + goff, 16)]
                acc = acc + plsc.load_gather(s2_v, [ta * 16 + tb])
            trm = term_v[pl.ds(poff + goff, 16)]
            gv = plsc.load_gather(m2_v, [trm * 16 + tok0])
            ms = ms + jnp.where(valid, gv, 0.0)
            logits = acc * (1.0 / T)
            mf = mask_v[pl.ds(poff + goff, 16)]
            logits = jnp.where(mf > 0.0, -1e9, logits)
            logits = jnp.where(valid, logits, -1e9)
            logit_v[pl.ds(goff, 16)] = logits
            score = logits + gum_v[pl.ds(poff + goff, 16)]
            score = jnp.where(valid, score, -3.0e38)
            upd = score > bs
            bs = jnp.where(upd, score, bs)
            bi = jnp.where(upd, gidx, bi)
            se = se + jnp.exp(logits)
            return bs, bi, se, ms

        bs0 = jnp.full((16,), -3.0e38, jnp.float32)
        bs, bi, se, msum = lax.fori_loop(
            0, CH, chunk_body, (bs0, zi, zf, zf))

        m = jnp.max(bs)
        sample = jnp.min(jnp.where(bs == m, bi, jnp.int32(1 << 30)))
        sumexp = jnp.sum(se)
        xv = zf + sumexp
        y = (plsc.bitcast(xv, jnp.int32).astype(jnp.float32)
             * (2.0 ** -23) - 127.0) * LN2
        for _ in range(3):
            y = y + xv * jnp.exp(-y) - 1.0
        lsv = plsc.load_gather(logit_v, [zi + sample])
        logp_v = lsv - y
        mse_v = 1000.0 / ((zf + jnp.sum(msum)) + MSE_BIAS)

        samp_f = (zi + sample).astype(jnp.float32)
        out_st[...] = jnp.where(iota == 0, samp_f,
                                jnp.where(iota == 1, logp_v,
                                          jnp.where(iota == 2, mse_v, 0.0)))
        pltpu.sync_copy(out_st, out_hbm.at[pl.ds(b * 16, 16)])


def kernel(dag_tokens, terminal_tokens, mask, emb_table, w, gumbel):
    tbl = jnp.zeros((16, D), jnp.float32).at[:V].set(emb_table.astype(jnp.float32))
    w2 = w.astype(jnp.float32).reshape(1, D)
    s2, m2 = pl.pallas_call(
        _tables_body,
        out_shape=(jax.ShapeDtypeStruct((16, 16), jnp.float32),
                   jax.ShapeDtypeStruct((16, 16), jnp.float32)),
    )(tbl, w2)

    mesh = plsc.VectorSubcoreMesh(core_axis_name="c", subcore_axis_name="s",
                                  num_cores=NC, num_subcores=NS)
    sc = pl.kernel(
        _sc_body,
        out_type=jax.ShapeDtypeStruct((B * 16,), jnp.float32),
        mesh=mesh,
        compiler_params=pltpu.CompilerParams(needs_layout_passes=False),
        scratch_types=[
            pltpu.VMEM((BPW * TG + 16,), jnp.int32),
            pltpu.VMEM((BPW * G + 16,), jnp.int32),
            pltpu.VMEM((BPW * G + 16,), jnp.float32),
            pltpu.VMEM((BPW * G + 16,), jnp.float32),
            pltpu.VMEM((GP,), jnp.float32),
            pltpu.VMEM((256,), jnp.float32),
            pltpu.VMEM((256,), jnp.float32),
            pltpu.VMEM((16,), jnp.float32),
            pltpu.SemaphoreType.DMA,
        ],
    )
    out = sc(
        dag_tokens.astype(jnp.int32).reshape(B * TG),
        terminal_tokens.astype(jnp.int32).reshape(B * G),
        mask.astype(jnp.float32).reshape(B * G),
        gumbel.astype(jnp.float32).reshape(B * G),
        s2.reshape(256),
        m2.reshape(256),
    )
    stats = out.reshape(B, 16)
    sample = stats[:, 0].astype(jnp.int32)
    return (sample, jnp.stack([stats[:, 1], stats[:, 2]]))
